# Initial kernel scaffold; baseline (speedup 1.0000x reference)
import jax
import jax.numpy as jnp
from jax.experimental import pallas as pl


def kernel(x, w_pool, b_pool, w1, b1, w2, b2):
    raise NotImplementedError("write your pallas kernel here")


# speedup vs baseline: 1.3806x; 1.3806x over previous
import functools

import jax
import jax.numpy as jnp
from jax.experimental import pallas as pl
from jax.experimental.pallas import tpu as pltpu

B, S, D = 4, 8192, 1024
HID = 512
NUM_OUT = 8
CS = 2048
NC = S // CS


def _router_kernel(x_ref, w_pool_ref, w1_ref, b1_ref, w2_ref, b2_ref,
                   out_ref, pooled_ref, m_ref, l_ref):
    b = pl.program_id(0)
    c = pl.program_id(1)

    @pl.when(c == 0)
    def _init():
        m_ref[0] = -jnp.inf
        l_ref[0] = 0.0

    x_blk = x_ref[0]
    s = jnp.dot(x_blk, w_pool_ref[...], preferred_element_type=jnp.float32)
    m_c = jnp.max(s)
    m_prev = m_ref[0]
    m_new = jnp.maximum(m_prev, m_c)
    m_ref[0] = m_new
    alpha = jnp.exp(m_prev - m_new)
    p = jnp.exp(s - m_new)
    l_ref[0] = l_ref[0] * alpha + jnp.sum(p)
    acc_c = jnp.dot(p.T, x_blk, preferred_element_type=jnp.float32)

    @pl.when(c == 0)
    def _first():
        pooled_ref[pl.ds(b, 1), :] = acc_c

    @pl.when(c > 0)
    def _rest():
        pooled_ref[pl.ds(b, 1), :] = pooled_ref[pl.ds(b, 1), :] * alpha + acc_c

    @pl.when(c == NC - 1)
    def _finish_batch():
        pooled_ref[pl.ds(b, 1), :] = pooled_ref[pl.ds(b, 1), :] / l_ref[0]

    @pl.when((b == B - 1) & (c == NC - 1))
    def _mlp():
        pooled = pooled_ref[...]
        h = jnp.dot(pooled, w1_ref[...], preferred_element_type=jnp.float32)
        h = jnp.maximum(h + b1_ref[...], 0.0)
        logits = jnp.dot(h, w2_ref[...], preferred_element_type=jnp.float32)
        logits = logits + b2_ref[...]

        col = jax.lax.broadcasted_iota(jnp.int32, (B, NUM_OUT), 1)
        m1 = jnp.max(logits, axis=1, keepdims=True)
        i1 = jnp.min(jnp.where(logits == m1, col, NUM_OUT), axis=1, keepdims=True)
        l2 = jnp.where(col == i1, -jnp.inf, logits)
        m2 = jnp.max(l2, axis=1, keepdims=True)
        i2 = jnp.min(jnp.where(l2 == m2, col, NUM_OUT), axis=1, keepdims=True)
        keep = (col == i1) | (col == i2)
        e = jnp.where(keep, jnp.exp(logits - m1), 0.0)
        out_ref[...] = e / jnp.sum(e, axis=1, keepdims=True)


@functools.partial(jax.jit, static_argnames=())
def kernel(x, w_pool, b_pool, w1, b1, w2, b2):
    del b_pool
    b1_2d = b1.reshape(1, HID)
    b2_2d = b2.reshape(1, NUM_OUT)
    return pl.pallas_call(
        _router_kernel,
        grid=(B, NC),
        in_specs=[
            pl.BlockSpec((1, CS, D), lambda b, c: (b, c, 0)),
            pl.BlockSpec((D, 1), lambda b, c: (0, 0)),
            pl.BlockSpec((D, HID), lambda b, c: (0, 0)),
            pl.BlockSpec((1, HID), lambda b, c: (0, 0)),
            pl.BlockSpec((HID, NUM_OUT), lambda b, c: (0, 0)),
            pl.BlockSpec((1, NUM_OUT), lambda b, c: (0, 0)),
        ],
        out_specs=pl.BlockSpec((B, NUM_OUT), lambda b, c: (0, 0)),
        out_shape=jax.ShapeDtypeStruct((B, NUM_OUT), jnp.float32),
        scratch_shapes=[
            pltpu.VMEM((B, D), jnp.float32),
            pltpu.SMEM((1,), jnp.float32),
            pltpu.SMEM((1,), jnp.float32),
        ],
    )(x, w_pool, w1, b1_2d, w2, b2_2d)


# speedup vs baseline: 1.6460x; 1.1922x over previous
import functools

import jax
import jax.numpy as jnp
from jax.experimental import pallas as pl
from jax.experimental.pallas import tpu as pltpu

B, S, D = 4, 8192, 1024
HID = 512
NUM_OUT = 8
CS = 2048
NC = S // CS


def _router_kernel(x_ref, w_pool_ref, w1_ref, b1_ref, w2_ref, b2_ref,
                   out_ref, pooled_ref, m_ref, l_ref):
    b = pl.program_id(0)
    c = pl.program_id(1)

    @pl.when(c == 0)
    def _init():
        m_ref[0] = -jnp.inf
        l_ref[0] = 0.0

    x_blk = x_ref[0]
    wp_row = w_pool_ref[...].reshape(1, D)
    s = jnp.sum(x_blk * wp_row, axis=1, keepdims=True)
    m_c = jnp.max(s)
    m_prev = m_ref[0]
    m_new = jnp.maximum(m_prev, m_c)
    m_ref[0] = m_new
    alpha = jnp.exp(m_prev - m_new)
    p = jnp.exp(s - m_new)
    l_ref[0] = l_ref[0] * alpha + jnp.sum(p)
    acc_c = jnp.sum(p * x_blk, axis=0, keepdims=True)

    @pl.when(c == 0)
    def _first():
        pooled_ref[pl.ds(b, 1), :] = acc_c

    @pl.when(c > 0)
    def _rest():
        pooled_ref[pl.ds(b, 1), :] = pooled_ref[pl.ds(b, 1), :] * alpha + acc_c

    @pl.when(c == NC - 1)
    def _finish_batch():
        pooled_ref[pl.ds(b, 1), :] = pooled_ref[pl.ds(b, 1), :] / l_ref[0]

    @pl.when((b == B - 1) & (c == NC - 1))
    def _mlp():
        pooled = pooled_ref[...]
        h = jnp.dot(pooled, w1_ref[...], preferred_element_type=jnp.float32)
        h = jnp.maximum(h + b1_ref[...], 0.0)
        logits = jnp.dot(h, w2_ref[...], preferred_element_type=jnp.float32)
        logits = logits + b2_ref[...]

        col = jax.lax.broadcasted_iota(jnp.int32, (B, NUM_OUT), 1)
        m1 = jnp.max(logits, axis=1, keepdims=True)
        i1 = jnp.min(jnp.where(logits == m1, col, NUM_OUT), axis=1, keepdims=True)
        l2 = jnp.where(col == i1, -jnp.inf, logits)
        m2 = jnp.max(l2, axis=1, keepdims=True)
        i2 = jnp.min(jnp.where(l2 == m2, col, NUM_OUT), axis=1, keepdims=True)
        keep = (col == i1) | (col == i2)
        e = jnp.where(keep, jnp.exp(logits - m1), 0.0)
        out_ref[...] = e / jnp.sum(e, axis=1, keepdims=True)


@functools.partial(jax.jit, static_argnames=())
def kernel(x, w_pool, b_pool, w1, b1, w2, b2):
    del b_pool
    b1_2d = b1.reshape(1, HID)
    b2_2d = b2.reshape(1, NUM_OUT)
    return pl.pallas_call(
        _router_kernel,
        grid=(B, NC),
        in_specs=[
            pl.BlockSpec((1, CS, D), lambda b, c: (b, c, 0)),
            pl.BlockSpec((D, 1), lambda b, c: (0, 0)),
            pl.BlockSpec((D, HID), lambda b, c: (0, 0)),
            pl.BlockSpec((1, HID), lambda b, c: (0, 0)),
            pl.BlockSpec((HID, NUM_OUT), lambda b, c: (0, 0)),
            pl.BlockSpec((1, NUM_OUT), lambda b, c: (0, 0)),
        ],
        out_specs=pl.BlockSpec((B, NUM_OUT), lambda b, c: (0, 0)),
        out_shape=jax.ShapeDtypeStruct((B, NUM_OUT), jnp.float32),
        scratch_shapes=[
            pltpu.VMEM((B, D), jnp.float32),
            pltpu.SMEM((1,), jnp.float32),
            pltpu.SMEM((1,), jnp.float32),
        ],
    )(x, w_pool, w1, b1_2d, w2, b2_2d)


# speedup vs baseline: 1.7428x; 1.0588x over previous
import functools

import jax
import jax.numpy as jnp
from jax.experimental import pallas as pl
from jax.experimental.pallas import tpu as pltpu

B, S, D = 4, 8192, 1024
HID = 512
NUM_OUT = 8
CS = 2048
NC = S // CS


def _router_kernel(x_ref, w_pool_ref, w1_ref, b1_ref, w2_ref, b2_ref,
                   out_ref, pooled_ref, m_ref, l_ref):
    b = pl.program_id(0)
    c = pl.program_id(1)

    @pl.when(c == 0)
    def _init():
        m_ref[0] = -jnp.inf
        l_ref[0] = 0.0

    x_blk = x_ref[0]
    wp_row = w_pool_ref[...].reshape(1, D)
    s = jnp.sum(x_blk * wp_row, axis=1, keepdims=True)
    m_c = jnp.max(s)
    m_prev = m_ref[0]
    m_new = jnp.maximum(m_prev, m_c)
    m_ref[0] = m_new
    alpha = jnp.exp(m_prev - m_new)
    p = jnp.exp(s - m_new)
    l_ref[0] = l_ref[0] * alpha + jnp.sum(p)
    acc_c = jnp.dot(p.T, x_blk, preferred_element_type=jnp.float32)

    @pl.when(c == 0)
    def _first():
        pooled_ref[pl.ds(b, 1), :] = acc_c

    @pl.when(c > 0)
    def _rest():
        pooled_ref[pl.ds(b, 1), :] = pooled_ref[pl.ds(b, 1), :] * alpha + acc_c

    @pl.when(c == NC - 1)
    def _finish_batch():
        pooled_ref[pl.ds(b, 1), :] = pooled_ref[pl.ds(b, 1), :] / l_ref[0]

    @pl.when((b == B - 1) & (c == NC - 1))
    def _mlp():
        pooled = pooled_ref[...]
        h = jnp.dot(pooled, w1_ref[...], preferred_element_type=jnp.float32)
        h = jnp.maximum(h + b1_ref[...], 0.0)
        logits = jnp.dot(h, w2_ref[...], preferred_element_type=jnp.float32)
        logits = logits + b2_ref[...]

        col = jax.lax.broadcasted_iota(jnp.int32, (B, NUM_OUT), 1)
        m1 = jnp.max(logits, axis=1, keepdims=True)
        i1 = jnp.min(jnp.where(logits == m1, col, NUM_OUT), axis=1, keepdims=True)
        l2 = jnp.where(col == i1, -jnp.inf, logits)
        m2 = jnp.max(l2, axis=1, keepdims=True)
        i2 = jnp.min(jnp.where(l2 == m2, col, NUM_OUT), axis=1, keepdims=True)
        keep = (col == i1) | (col == i2)
        e = jnp.where(keep, jnp.exp(logits - m1), 0.0)
        out_ref[...] = e / jnp.sum(e, axis=1, keepdims=True)


@functools.partial(jax.jit, static_argnames=())
def kernel(x, w_pool, b_pool, w1, b1, w2, b2):
    del b_pool
    b1_2d = b1.reshape(1, HID)
    b2_2d = b2.reshape(1, NUM_OUT)
    return pl.pallas_call(
        _router_kernel,
        grid=(B, NC),
        in_specs=[
            pl.BlockSpec((1, CS, D), lambda b, c: (b, c, 0)),
            pl.BlockSpec((D, 1), lambda b, c: (0, 0)),
            pl.BlockSpec((D, HID), lambda b, c: (0, 0)),
            pl.BlockSpec((1, HID), lambda b, c: (0, 0)),
            pl.BlockSpec((HID, NUM_OUT), lambda b, c: (0, 0)),
            pl.BlockSpec((1, NUM_OUT), lambda b, c: (0, 0)),
        ],
        out_specs=pl.BlockSpec((B, NUM_OUT), lambda b, c: (0, 0)),
        out_shape=jax.ShapeDtypeStruct((B, NUM_OUT), jnp.float32),
        scratch_shapes=[
            pltpu.VMEM((B, D), jnp.float32),
            pltpu.SMEM((1,), jnp.float32),
            pltpu.SMEM((1,), jnp.float32),
        ],
    )(x, w_pool, w1, b1_2d, w2, b2_2d)


# speedup vs baseline: 1.7867x; 1.0252x over previous
import functools

import jax
import jax.numpy as jnp
from jax.experimental import pallas as pl
from jax.experimental.pallas import tpu as pltpu

B, S, D = 4, 8192, 1024
HID = 512
NUM_OUT = 8
CS = 4096
NC = S // CS


def _router_kernel(x_ref, w_pool_ref, w1_ref, b1_ref, w2_ref, b2_ref,
                   out_ref, pooled_ref, m_ref, l_ref):
    b = pl.program_id(0)
    c = pl.program_id(1)

    @pl.when(c == 0)
    def _init():
        m_ref[0] = -jnp.inf
        l_ref[0] = 0.0

    x_blk = x_ref[0]
    wp_row = w_pool_ref[...].reshape(1, D)
    s = jnp.sum(x_blk * wp_row, axis=1, keepdims=True)
    m_c = jnp.max(s)
    m_prev = m_ref[0]
    m_new = jnp.maximum(m_prev, m_c)
    m_ref[0] = m_new
    alpha = jnp.exp(m_prev - m_new)
    p = jnp.exp(s - m_new)
    l_ref[0] = l_ref[0] * alpha + jnp.sum(p)
    acc_c = jnp.dot(p.T, x_blk, preferred_element_type=jnp.float32)

    @pl.when(c == 0)
    def _first():
        pooled_ref[pl.ds(b, 1), :] = acc_c

    @pl.when(c > 0)
    def _rest():
        pooled_ref[pl.ds(b, 1), :] = pooled_ref[pl.ds(b, 1), :] * alpha + acc_c

    @pl.when(c == NC - 1)
    def _finish_batch():
        pooled_ref[pl.ds(b, 1), :] = pooled_ref[pl.ds(b, 1), :] / l_ref[0]

    @pl.when((b == B - 1) & (c == NC - 1))
    def _mlp():
        pooled = pooled_ref[...]
        h = jnp.dot(pooled, w1_ref[...], preferred_element_type=jnp.float32)
        h = jnp.maximum(h + b1_ref[...], 0.0)
        logits = jnp.dot(h, w2_ref[...], preferred_element_type=jnp.float32)
        logits = logits + b2_ref[...]

        col = jax.lax.broadcasted_iota(jnp.int32, (B, NUM_OUT), 1)
        m1 = jnp.max(logits, axis=1, keepdims=True)
        i1 = jnp.min(jnp.where(logits == m1, col, NUM_OUT), axis=1, keepdims=True)
        l2 = jnp.where(col == i1, -jnp.inf, logits)
        m2 = jnp.max(l2, axis=1, keepdims=True)
        i2 = jnp.min(jnp.where(l2 == m2, col, NUM_OUT), axis=1, keepdims=True)
        keep = (col == i1) | (col == i2)
        e = jnp.where(keep, jnp.exp(logits - m1), 0.0)
        out_ref[...] = e / jnp.sum(e, axis=1, keepdims=True)


@functools.partial(jax.jit, static_argnames=())
def kernel(x, w_pool, b_pool, w1, b1, w2, b2):
    del b_pool
    b1_2d = b1.reshape(1, HID)
    b2_2d = b2.reshape(1, NUM_OUT)
    return pl.pallas_call(
        _router_kernel,
        grid=(B, NC),
        in_specs=[
            pl.BlockSpec((1, CS, D), lambda b, c: (b, c, 0)),
            pl.BlockSpec((D, 1), lambda b, c: (0, 0)),
            pl.BlockSpec((D, HID), lambda b, c: (0, 0)),
            pl.BlockSpec((1, HID), lambda b, c: (0, 0)),
            pl.BlockSpec((HID, NUM_OUT), lambda b, c: (0, 0)),
            pl.BlockSpec((1, NUM_OUT), lambda b, c: (0, 0)),
        ],
        out_specs=pl.BlockSpec((B, NUM_OUT), lambda b, c: (0, 0)),
        out_shape=jax.ShapeDtypeStruct((B, NUM_OUT), jnp.float32),
        scratch_shapes=[
            pltpu.VMEM((B, D), jnp.float32),
            pltpu.SMEM((1,), jnp.float32),
            pltpu.SMEM((1,), jnp.float32),
        ],
    )(x, w_pool, w1, b1_2d, w2, b2_2d)
